# baseline (speedup 1.0000x reference)
import functools

import jax
import jax.numpy as jnp
from jax import lax
from jax.experimental import pallas as pl
from jax.experimental.pallas import tpu as pltpu
from jax.experimental.pallas import tpu_sc as plsc

_TOP_K = 10
_TAU = 1e-4
_EPS = 1e-8
_N = 2048
_B = 8
_CAP = 64
_TGT = 48
_BS_ITERS = 16
_PK = 3 * _CAP


def _shift_right(p, sh):
    z = jnp.zeros((p.shape[0], sh), p.dtype)
    return jnp.concatenate([z, p[:, : p.shape[1] - sh]], axis=1)


def _k1_body(r_ref, out_ref):
    inv_tau = jnp.float32(1.0 / _TAU)
    x = r_ref[...]

    lo = jnp.min(x, axis=1, keepdims=True) - 1.0
    hi = jnp.max(x, axis=1, keepdims=True)

    def bs(_, carry):
        lo, hi = carry
        mid = 0.5 * (lo + hi)
        cnt = jnp.sum((x > mid).astype(jnp.float32), axis=1, keepdims=True)
        pred = cnt >= _TGT
        return jnp.where(pred, mid, lo), jnp.where(pred, hi, mid)

    lo, hi = jax.lax.fori_loop(0, _BS_ITERS, bs, (lo, hi))
    mask = (x > lo).astype(jnp.float32)

    p = mask
    sh = 1
    while sh < _N:
        p = p + _shift_right(p, sh)
        sh *= 2
    cnt_all = p[:, _N - 1 : _N]

    iota_n = jax.lax.broadcasted_iota(jnp.int32, (_B, _N), 1)
    neg = jnp.float32(-jnp.inf)

    def kstep(_, carry):
        vals, ind = carry
        mx = jnp.max(vals, axis=1, keepdims=True)
        hit = vals == mx
        first = jnp.min(jnp.where(hit, iota_n, _N), axis=1, keepdims=True)
        oh = jnp.where(iota_n == first, 1.0, 0.0)
        return jnp.where(oh > 0, neg, vals), ind + oh

    _, ind = jax.lax.fori_loop(
        0, _TOP_K, kstep, (x, jnp.zeros((_B, _N), jnp.float32)))

    x_rep = jnp.concatenate(
        [jnp.broadcast_to(x[b : b + 1, :], (_CAP, _N)) for b in range(_B)], 0)
    p_rep = jnp.concatenate(
        [jnp.broadcast_to(p[b : b + 1, :], (_CAP, _N)) for b in range(_B)], 0)
    m_rep = jnp.concatenate(
        [jnp.broadcast_to(mask[b : b + 1, :], (_CAP, _N)) for b in range(_B)],
        0)
    i_rep = jnp.concatenate(
        [jnp.broadcast_to(ind[b : b + 1, :], (_CAP, _N)) for b in range(_B)],
        0)
    cnt_rep = jnp.concatenate(
        [jnp.broadcast_to(cnt_all[b : b + 1, :], (_CAP, 1)) for b in range(_B)],
        0)
    nrows = _B * _CAP
    iota_r = jax.lax.broadcasted_iota(jnp.int32, (nrows, 1), 0)
    slot = (iota_r - (iota_r >> 6 << 6)).astype(jnp.float32)
    bt = jnp.where((p_rep - 1.0 == slot) & (m_rep > 0), 1.0, 0.0)
    iota_nf = jax.lax.broadcasted_iota(
        jnp.int32, (nrows, _N), 1).astype(jnp.float32)
    cand = jnp.sum(bt * x_rep, axis=1, keepdims=True)
    idxc = jnp.sum(bt * iota_nf, axis=1, keepdims=True)
    indc = jnp.sum(bt * i_rep, axis=1, keepdims=True)
    d = (cand - x_rep) * inv_tau
    rank = jnp.sum(jax.nn.sigmoid(d), axis=1, keepdims=True) + 0.5
    valid = slot < cnt_rep
    wcol = jnp.where(valid, jax.nn.sigmoid(rank - (_N - _TOP_K)), 0.0)

    iota_rc = jax.lax.broadcasted_iota(jnp.int32, (nrows, _CAP), 0)
    iota_cc = jax.lax.broadcasted_iota(jnp.int32, (nrows, _CAP), 1)
    oh_mod = jnp.where(iota_rc - (iota_rc >> 6 << 6) == iota_cc, 1.0, 0.0)
    t_all = jnp.concatenate(
        [idxc * oh_mod, wcol * oh_mod, indc * oh_mod], axis=1)
    for b in range(_B):
        out_ref[b : b + 1, :] = jnp.sum(
            t_all[b * _CAP : (b + 1) * _CAP, :], axis=0, keepdims=True)


def _k1_call(R):
    return pl.pallas_call(
        _k1_body,
        in_specs=[pl.BlockSpec((_B, _N), lambda: (0, 0))],
        out_specs=pl.BlockSpec((_B, _PK), lambda: (0, 0)),
        out_shape=jax.ShapeDtypeStruct((_B, _PK), jnp.float32),
    )(R)


def _iota16():
    return jax.lax.broadcasted_iota(jnp.int32, (16,), 0)


def _splat16(v):
    return jnp.full((16,), v, jnp.int32)


def _k2_body(m_hbm, pk_hbm, out_hbm,
             pk_v, idx_v, ridx_v, rows_v, part_v, all_v, outst_v, shared,
             sem):
    c = lax.axis_index("c")
    s = lax.axis_index("s")
    a_global = c * 4 + s // 4
    q = s % 4

    pltpu.sync_copy(pk_hbm.at[a_global], pk_v)
    for ch in range(4):
        idx_v[pl.ds(ch * 16, 16)] = pk_v[pl.ds(ch * 16, 16)].astype(jnp.int32)

    ridx_v[...] = plsc.load_gather(idx_v, [q * 16 + _iota16()])
    pltpu.async_copy(m_hbm.at[ridx_v], rows_v, sem).wait()

    wap = [pk_v[pl.ds(64 + ch * 16, 16)] for ch in range(4)]
    win = [pk_v[pl.ds(128 + ch * 16, 16)] for ch in range(4)]
    colidx = [plsc.load_gather(idx_v, [ch * 16 + _iota16()])
              for ch in range(4)]

    zero = jnp.zeros((16,), jnp.float32)
    acc_ap = zero
    acc_in = zero
    for r in range(16):
        w_ap_r = plsc.load_gather(pk_v, [_splat16(64 + q * 16 + r)])
        w_in_r = plsc.load_gather(pk_v, [_splat16(128 + q * 16 + r)])
        row_sel = _splat16(r)
        for ch in range(4):
            g = plsc.load_gather(rows_v, [row_sel, colidx[ch]])
            acc_ap = acc_ap + w_ap_r * (g * wap[ch])
            acc_in = acc_in + w_in_r * (g * win[ch])

    qz = jnp.where(jnp.full((16,), q, jnp.int32) == 0, 1.0, 0.0)
    ws_ap = (wap[0] + wap[1] + wap[2] + wap[3]) * qz
    wss_ap = (wap[0] * wap[0] + wap[1] * wap[1]
              + wap[2] * wap[2] + wap[3] * wap[3]) * qz
    ws_in = (win[0] + win[1] + win[2] + win[3]) * qz
    wss_in = (win[0] * win[0] + win[1] * win[1]
              + win[2] * win[2] + win[3] * win[3]) * qz

    part_v[0, :] = acc_ap
    part_v[1, :] = acc_in
    part_v[2, :] = ws_ap
    part_v[3, :] = wss_ap
    part_v[4, :] = ws_in
    part_v[5, :] = wss_in
    pltpu.sync_copy(part_v, shared.at[s])
    plsc.subcore_barrier()

    @pl.when(s == 0)
    def _finalize():
        pltpu.sync_copy(shared, all_v)
        iota = _iota16()
        acc = [zero] * 6
        for a_l in range(4):
            lane = jnp.where(iota == c * 4 + a_l, 1.0, 0.0)
            for row_i in range(6):
                v = (all_v[4 * a_l + 0, row_i, :]
                     + all_v[4 * a_l + 1, row_i, :]
                     + all_v[4 * a_l + 2, row_i, :]
                     + all_v[4 * a_l + 3, row_i, :])
                acc[row_i] = acc[row_i] + jnp.sum(v, axis=0) * lane
        num_ap, num_in, ws1, wss1, ws2, wss2 = acc
        den1 = ws1 * ws1 - wss1
        den2 = ws2 * ws2 - wss2
        out_ap = jnp.where(den1 == 0.0, 0.0, num_ap / (den1 + _EPS))
        out_in = jnp.where(den2 == 0.0, 0.0, num_in / (den2 + _EPS))
        outst_v[0, :] = out_ap
        outst_v[1, :] = out_in
        pltpu.sync_copy(outst_v, out_hbm.at[pl.ds(c * 2, 2)])


@functools.cache
def _k2_kernel():
    return pl.kernel(
        _k2_body,
        mesh=plsc.VectorSubcoreMesh(
            core_axis_name="c", subcore_axis_name="s"),
        compiler_params=pltpu.CompilerParams(
            needs_layout_passes=False, use_tc_tiling_on_sc=False),
        out_type=jax.ShapeDtypeStruct((4, 16), jnp.float32),
        scratch_types=[
            pltpu.VMEM((_PK,), jnp.float32),
            pltpu.VMEM((_CAP,), jnp.int32),
            pltpu.VMEM((16,), jnp.int32),
            pltpu.VMEM((16, _N), jnp.float32),
            pltpu.VMEM((6, 16), jnp.float32),
            pltpu.VMEM((16, 6, 16), jnp.float32),
            pltpu.VMEM((2, 16), jnp.float32),
            pltpu.VMEM_SHARED((16, 6, 16), jnp.float32),
            pltpu.SemaphoreType.DMA,
        ],
    )


def _k2_call(m_hbm, pk_hbm):
    return _k2_kernel()(m_hbm, pk_hbm)


def kernel(R, dist_mat):
    packed = _k1_call(R)
    out4 = _k2_call(dist_mat, packed)
    approx = jnp.concatenate([out4[0, :4], out4[2, 4:8]])
    real = jnp.concatenate([out4[1, :4], out4[3, 4:8]])
    return approx, real


# speedup vs baseline: 1.2794x; 1.2794x over previous
import functools

import jax
import jax.numpy as jnp
from jax import lax
from jax.experimental import pallas as pl
from jax.experimental.pallas import tpu as pltpu
from jax.experimental.pallas import tpu_sc as plsc

_TOP_K = 10
_TAU = 1e-4
_EPS = 1e-8
_N = 2048
_B = 8
_CAP = 64
_TGT = 48
_BS_ITERS = 16
_PK = 3 * _CAP


def _shift_right(p, sh):
    z = jnp.zeros((p.shape[0], sh), p.dtype)
    return jnp.concatenate([z, p[:, : p.shape[1] - sh]], axis=1)


def _k1_body(r_ref, out_ref):
    inv_tau = jnp.float32(1.0 / _TAU)
    x = r_ref[...]

    lo = jnp.min(x, axis=1, keepdims=True) - 1.0
    hi = jnp.max(x, axis=1, keepdims=True)

    def bs(_, carry):
        lo, hi = carry
        mid = 0.5 * (lo + hi)
        cnt = jnp.sum((x > mid).astype(jnp.float32), axis=1, keepdims=True)
        pred = cnt >= _TGT
        return jnp.where(pred, mid, lo), jnp.where(pred, hi, mid)

    lo, hi = jax.lax.fori_loop(0, _BS_ITERS, bs, (lo, hi))
    mask = (x > lo).astype(jnp.float32)

    p = mask
    sh = 1
    while sh < _N:
        p = p + _shift_right(p, sh)
        sh *= 2
    cnt_all = p[:, _N - 1 : _N]

    iota_n = jax.lax.broadcasted_iota(jnp.int32, (_B, _N), 1)
    neg = jnp.float32(-jnp.inf)

    def kstep(_, carry):
        vals, ind = carry
        mx = jnp.max(vals, axis=1, keepdims=True)
        hit = vals == mx
        first = jnp.min(jnp.where(hit, iota_n, _N), axis=1, keepdims=True)
        oh = jnp.where(iota_n == first, 1.0, 0.0)
        return jnp.where(oh > 0, neg, vals), ind + oh

    _, ind = jax.lax.fori_loop(
        0, _TOP_K, kstep, (x, jnp.zeros((_B, _N), jnp.float32)))

    x_rep = jnp.concatenate(
        [jnp.broadcast_to(x[b : b + 1, :], (_CAP, _N)) for b in range(_B)], 0)
    p_rep = jnp.concatenate(
        [jnp.broadcast_to(p[b : b + 1, :], (_CAP, _N)) for b in range(_B)], 0)
    m_rep = jnp.concatenate(
        [jnp.broadcast_to(mask[b : b + 1, :], (_CAP, _N)) for b in range(_B)],
        0)
    i_rep = jnp.concatenate(
        [jnp.broadcast_to(ind[b : b + 1, :], (_CAP, _N)) for b in range(_B)],
        0)
    cnt_rep = jnp.concatenate(
        [jnp.broadcast_to(cnt_all[b : b + 1, :], (_CAP, 1)) for b in range(_B)],
        0)
    nrows = _B * _CAP
    iota_r = jax.lax.broadcasted_iota(jnp.int32, (nrows, 1), 0)
    slot = (iota_r - (iota_r >> 6 << 6)).astype(jnp.float32)
    bt = jnp.where((p_rep - 1.0 == slot) & (m_rep > 0), 1.0, 0.0)
    iota_nf = jax.lax.broadcasted_iota(
        jnp.int32, (nrows, _N), 1).astype(jnp.float32)
    cand = jnp.sum(bt * x_rep, axis=1, keepdims=True)
    idxc = jnp.sum(bt * iota_nf, axis=1, keepdims=True)
    indc = jnp.sum(bt * i_rep, axis=1, keepdims=True)
    d = (cand - x_rep) * inv_tau
    rank = jnp.sum(jax.nn.sigmoid(d), axis=1, keepdims=True) + 0.5
    valid = slot < cnt_rep
    wcol = jnp.where(valid, jax.nn.sigmoid(rank - (_N - _TOP_K)), 0.0)

    iota_rc = jax.lax.broadcasted_iota(jnp.int32, (nrows, _CAP), 0)
    iota_cc = jax.lax.broadcasted_iota(jnp.int32, (nrows, _CAP), 1)
    oh_mod = jnp.where(iota_rc - (iota_rc >> 6 << 6) == iota_cc, 1.0, 0.0)
    t_all = jnp.concatenate(
        [idxc * oh_mod, wcol * oh_mod, indc * oh_mod], axis=1)
    for b in range(_B):
        out_ref[b : b + 1, :] = jnp.sum(
            t_all[b * _CAP : (b + 1) * _CAP, :], axis=0, keepdims=True)


def _k1_call(R):
    return pl.pallas_call(
        _k1_body,
        in_specs=[pl.BlockSpec((_B, _N), lambda: (0, 0))],
        out_specs=pl.BlockSpec((_B, _PK), lambda: (0, 0)),
        out_shape=jax.ShapeDtypeStruct((_B, _PK), jnp.float32),
    )(R)


def _iota16():
    return jax.lax.broadcasted_iota(jnp.int32, (16,), 0)


def _splat16(v):
    return jnp.full((16,), v, jnp.int32)


def _k2_body(m_hbm, pk_hbm, out_hbm,
             pk_v, idx_v, ridx_v, rows_v, part_v, all_v, outst_v, shared,
             sem):
    c = lax.axis_index("c")
    s = lax.axis_index("s")
    a_global = c * 4 + s // 4
    q = s % 4

    pltpu.sync_copy(pk_hbm.at[a_global], pk_v)
    for ch in range(4):
        idx_v[pl.ds(ch * 16, 16)] = pk_v[pl.ds(ch * 16, 16)].astype(jnp.int32)

    ridx_v[...] = plsc.load_gather(idx_v, [q * 16 + _iota16()])
    pltpu.async_copy(m_hbm.at[ridx_v], rows_v, sem).wait()

    wap = [pk_v[pl.ds(64 + ch * 16, 16)] for ch in range(4)]
    win = [pk_v[pl.ds(128 + ch * 16, 16)] for ch in range(4)]
    colidx = [plsc.load_gather(idx_v, [ch * 16 + _iota16()])
              for ch in range(4)]

    zero = jnp.zeros((16,), jnp.float32)
    acc_ap = zero
    acc_in = zero
    for r in range(16):
        w_ap_r = plsc.load_gather(pk_v, [_splat16(64 + q * 16 + r)])
        w_in_r = plsc.load_gather(pk_v, [_splat16(128 + q * 16 + r)])
        row_sel = _splat16(r)
        for ch in range(4):
            g = plsc.load_gather(rows_v, [row_sel, colidx[ch]])
            acc_ap = acc_ap + w_ap_r * (g * wap[ch])
            acc_in = acc_in + w_in_r * (g * win[ch])

    qz = jnp.where(jnp.full((16,), q, jnp.int32) == 0, 1.0, 0.0)
    ws_ap = (wap[0] + wap[1] + wap[2] + wap[3]) * qz
    wss_ap = (wap[0] * wap[0] + wap[1] * wap[1]
              + wap[2] * wap[2] + wap[3] * wap[3]) * qz
    ws_in = (win[0] + win[1] + win[2] + win[3]) * qz
    wss_in = (win[0] * win[0] + win[1] * win[1]
              + win[2] * win[2] + win[3] * win[3]) * qz

    vals = (acc_ap, acc_in, ws_ap, wss_ap, ws_in, wss_in, zero, zero)
    for k, v in enumerate(vals):
        part_v[pl.ds(16 * k, 16)] = v
    pltpu.sync_copy(part_v, shared.at[s])
    plsc.subcore_barrier()

    @pl.when(s == 0)
    def _finalize():
        pltpu.sync_copy(shared, all_v)
        iota = _iota16()
        acc = [zero] * 6
        for a_l in range(4):
            lane = jnp.where(iota == c * 4 + a_l, 1.0, 0.0)
            for row_i in range(6):
                v = (all_v[4 * a_l + 0, pl.ds(16 * row_i, 16)]
                     + all_v[4 * a_l + 1, pl.ds(16 * row_i, 16)]
                     + all_v[4 * a_l + 2, pl.ds(16 * row_i, 16)]
                     + all_v[4 * a_l + 3, pl.ds(16 * row_i, 16)])
                acc[row_i] = acc[row_i] + jnp.sum(v, axis=0) * lane
        num_ap, num_in, ws1, wss1, ws2, wss2 = acc
        den1 = ws1 * ws1 - wss1
        den2 = ws2 * ws2 - wss2
        out_ap = jnp.where(den1 == 0.0, 0.0, num_ap / (den1 + _EPS))
        out_in = jnp.where(den2 == 0.0, 0.0, num_in / (den2 + _EPS))
        for k in range(8):
            outst_v[0, pl.ds(16 * k, 16)] = out_ap if k == 0 else zero
            outst_v[1, pl.ds(16 * k, 16)] = out_in if k == 0 else zero
        pltpu.sync_copy(outst_v, out_hbm.at[pl.ds(c * 2, 2)])


@functools.cache
def _k2_kernel():
    return pl.kernel(
        _k2_body,
        mesh=plsc.VectorSubcoreMesh(
            core_axis_name="c", subcore_axis_name="s"),
        compiler_params=pltpu.CompilerParams(needs_layout_passes=False),
        out_type=jax.ShapeDtypeStruct((4, 128), jnp.float32),
        scratch_types=[
            pltpu.VMEM((_PK,), jnp.float32),
            pltpu.VMEM((_CAP,), jnp.int32),
            pltpu.VMEM((16,), jnp.int32),
            pltpu.VMEM((16, _N), jnp.float32),
            pltpu.VMEM((128,), jnp.float32),
            pltpu.VMEM((16, 128), jnp.float32),
            pltpu.VMEM((2, 128), jnp.float32),
            pltpu.VMEM_SHARED((16, 128), jnp.float32),
            pltpu.SemaphoreType.DMA,
        ],
    )


def _k2_call(m_hbm, pk_hbm):
    return _k2_kernel()(m_hbm, pk_hbm)


def kernel(R, dist_mat):
    packed = _k1_call(R)
    out4 = _k2_call(dist_mat, packed)
    approx = jnp.concatenate([out4[0, :4], out4[2, 4:8]])
    real = jnp.concatenate([out4[1, :4], out4[3, 4:8]])
    return approx, real


# speedup vs baseline: 1.2800x; 1.0005x over previous
import functools

import jax
import jax.numpy as jnp
from jax import lax
from jax.experimental import pallas as pl
from jax.experimental.pallas import tpu as pltpu
from jax.experimental.pallas import tpu_sc as plsc

_TOP_K = 10
_TAU = 1e-4
_EPS = 1e-8
_N = 2048
_B = 8
_CAP = 64
_TGT = 48
_BS_ITERS = 16
_PK = 3 * _CAP


def _shift_right(p, sh):
    z = jnp.zeros((p.shape[0], sh), p.dtype)
    return jnp.concatenate([z, p[:, : p.shape[1] - sh]], axis=1)


def _k1_body(r_ref, out_ref):
    inv_tau = jnp.float32(1.0 / _TAU)
    x = r_ref[...]

    lo = jnp.min(x, axis=1, keepdims=True) - 1.0
    hi = jnp.max(x, axis=1, keepdims=True)

    def bs(_, carry):
        lo, hi = carry
        mid = 0.5 * (lo + hi)
        cnt = jnp.sum((x > mid).astype(jnp.float32), axis=1, keepdims=True)
        pred = cnt >= _TGT
        return jnp.where(pred, mid, lo), jnp.where(pred, hi, mid)

    lo, hi = jax.lax.fori_loop(0, _BS_ITERS, bs, (lo, hi))
    mask = (x > lo).astype(jnp.float32)

    p = mask
    sh = 1
    while sh < _N:
        p = p + _shift_right(p, sh)
        sh *= 2
    cnt_all = p[:, _N - 1 : _N]

    iota_n = jax.lax.broadcasted_iota(jnp.int32, (_B, _N), 1)
    neg = jnp.float32(-jnp.inf)

    def kstep(_, carry):
        vals, ind = carry
        mx = jnp.max(vals, axis=1, keepdims=True)
        hit = vals == mx
        first = jnp.min(jnp.where(hit, iota_n, _N), axis=1, keepdims=True)
        oh = jnp.where(iota_n == first, 1.0, 0.0)
        return jnp.where(oh > 0, neg, vals), ind + oh

    _, ind = jax.lax.fori_loop(
        0, _TOP_K, kstep, (x, jnp.zeros((_B, _N), jnp.float32)))

    x_rep = jnp.concatenate(
        [jnp.broadcast_to(x[b : b + 1, :], (_CAP, _N)) for b in range(_B)], 0)
    p_rep = jnp.concatenate(
        [jnp.broadcast_to(p[b : b + 1, :], (_CAP, _N)) for b in range(_B)], 0)
    m_rep = jnp.concatenate(
        [jnp.broadcast_to(mask[b : b + 1, :], (_CAP, _N)) for b in range(_B)],
        0)
    i_rep = jnp.concatenate(
        [jnp.broadcast_to(ind[b : b + 1, :], (_CAP, _N)) for b in range(_B)],
        0)
    cnt_rep = jnp.concatenate(
        [jnp.broadcast_to(cnt_all[b : b + 1, :], (_CAP, 1)) for b in range(_B)],
        0)
    nrows = _B * _CAP
    iota_r = jax.lax.broadcasted_iota(jnp.int32, (nrows, 1), 0)
    slot = (iota_r - (iota_r >> 6 << 6)).astype(jnp.float32)
    bt = jnp.where((p_rep - 1.0 == slot) & (m_rep > 0), 1.0, 0.0)
    iota_nf = jax.lax.broadcasted_iota(
        jnp.int32, (nrows, _N), 1).astype(jnp.float32)
    cand = jnp.sum(bt * x_rep, axis=1, keepdims=True)
    idxc = jnp.sum(bt * iota_nf, axis=1, keepdims=True)
    indc = jnp.sum(bt * i_rep, axis=1, keepdims=True)
    d = (cand - x_rep) * inv_tau
    rank = jnp.sum(jax.nn.sigmoid(d), axis=1, keepdims=True) + 0.5
    valid = slot < cnt_rep
    wcol = jnp.where(valid, jax.nn.sigmoid(rank - (_N - _TOP_K)), 0.0)

    iota_rc = jax.lax.broadcasted_iota(jnp.int32, (nrows, _CAP), 0)
    iota_cc = jax.lax.broadcasted_iota(jnp.int32, (nrows, _CAP), 1)
    oh_mod = jnp.where(iota_rc - (iota_rc >> 6 << 6) == iota_cc, 1.0, 0.0)
    t_all = jnp.concatenate(
        [idxc * oh_mod, wcol * oh_mod, indc * oh_mod], axis=1)
    for b in range(_B):
        out_ref[b : b + 1, :] = jnp.sum(
            t_all[b * _CAP : (b + 1) * _CAP, :], axis=0, keepdims=True)


def _k1_call(R):
    return pl.pallas_call(
        _k1_body,
        in_specs=[pl.BlockSpec((_B, _N), lambda: (0, 0))],
        out_specs=pl.BlockSpec((_B, _PK), lambda: (0, 0)),
        out_shape=jax.ShapeDtypeStruct((_B, _PK), jnp.float32),
    )(R)


def _iota16():
    return jax.lax.broadcasted_iota(jnp.int32, (16,), 0)


def _splat16(v):
    return jnp.full((16,), v, jnp.int32)


def _k2_body(m_hbm, pk_hbm, out_hbm,
             pk_v, idx_v, ridx_v, rows_v, part_v, all_v, outst_v, shared,
             sem):
    s = lax.axis_index("s")
    a_global = s // 2
    q = s % 2

    pltpu.sync_copy(pk_hbm.at[a_global], pk_v)
    for ch in range(4):
        idx_v[pl.ds(ch * 16, 16)] = pk_v[pl.ds(ch * 16, 16)].astype(jnp.int32)

    ridx_v[pl.ds(0, 16)] = plsc.load_gather(idx_v, [q * 32 + _iota16()])
    ridx_v[pl.ds(16, 16)] = plsc.load_gather(idx_v, [q * 32 + 16 + _iota16()])
    pltpu.async_copy(m_hbm.at[ridx_v], rows_v, sem).wait()

    wap = [pk_v[pl.ds(64 + ch * 16, 16)] for ch in range(4)]
    win = [pk_v[pl.ds(128 + ch * 16, 16)] for ch in range(4)]
    colidx = [plsc.load_gather(idx_v, [ch * 16 + _iota16()])
              for ch in range(4)]

    zero = jnp.zeros((16,), jnp.float32)
    acc_ap = zero
    acc_in = zero
    for r in range(32):
        w_ap_r = plsc.load_gather(pk_v, [_splat16(64 + q * 32 + r)])
        w_in_r = plsc.load_gather(pk_v, [_splat16(128 + q * 32 + r)])
        row_sel = _splat16(r)
        for ch in range(4):
            g = plsc.load_gather(rows_v, [row_sel, colidx[ch]])
            acc_ap = acc_ap + w_ap_r * (g * wap[ch])
            acc_in = acc_in + w_in_r * (g * win[ch])

    qz = jnp.where(jnp.full((16,), q, jnp.int32) == 0, 1.0, 0.0)
    ws_ap = (wap[0] + wap[1] + wap[2] + wap[3]) * qz
    wss_ap = (wap[0] * wap[0] + wap[1] * wap[1]
              + wap[2] * wap[2] + wap[3] * wap[3]) * qz
    ws_in = (win[0] + win[1] + win[2] + win[3]) * qz
    wss_in = (win[0] * win[0] + win[1] * win[1]
              + win[2] * win[2] + win[3] * win[3]) * qz

    vals = (acc_ap, acc_in, ws_ap, wss_ap, ws_in, wss_in, zero, zero)
    for k, v in enumerate(vals):
        part_v[pl.ds(16 * k, 16)] = v
    pltpu.sync_copy(part_v, shared.at[s])
    plsc.subcore_barrier()

    @pl.when(s == 0)
    def _finalize():
        pltpu.sync_copy(shared, all_v)
        iota = _iota16()
        acc = [zero] * 6
        for a_l in range(8):
            lane = jnp.where(iota == a_l, 1.0, 0.0)
            for row_i in range(6):
                v = (all_v[2 * a_l + 0, pl.ds(16 * row_i, 16)]
                     + all_v[2 * a_l + 1, pl.ds(16 * row_i, 16)])
                acc[row_i] = acc[row_i] + jnp.sum(v, axis=0) * lane
        num_ap, num_in, ws1, wss1, ws2, wss2 = acc
        den1 = ws1 * ws1 - wss1
        den2 = ws2 * ws2 - wss2
        out_ap = jnp.where(den1 == 0.0, 0.0, num_ap / (den1 + _EPS))
        out_in = jnp.where(den2 == 0.0, 0.0, num_in / (den2 + _EPS))
        for k in range(8):
            outst_v[0, pl.ds(16 * k, 16)] = out_ap if k == 0 else zero
            outst_v[1, pl.ds(16 * k, 16)] = out_in if k == 0 else zero
        pltpu.sync_copy(outst_v, out_hbm)


@functools.cache
def _k2_kernel():
    return pl.kernel(
        _k2_body,
        mesh=plsc.VectorSubcoreMesh(
            core_axis_name="c", subcore_axis_name="s", num_cores=1),
        compiler_params=pltpu.CompilerParams(needs_layout_passes=False),
        out_type=jax.ShapeDtypeStruct((2, 128), jnp.float32),
        scratch_types=[
            pltpu.VMEM((_PK,), jnp.float32),
            pltpu.VMEM((_CAP,), jnp.int32),
            pltpu.VMEM((32,), jnp.int32),
            pltpu.VMEM((32, _N), jnp.float32),
            pltpu.VMEM((128,), jnp.float32),
            pltpu.VMEM((16, 128), jnp.float32),
            pltpu.VMEM((2, 128), jnp.float32),
            pltpu.VMEM_SHARED((16, 128), jnp.float32),
            pltpu.SemaphoreType.DMA,
        ],
    )


def _k2_call(m_hbm, pk_hbm):
    return _k2_kernel()(m_hbm, pk_hbm)


def kernel(R, dist_mat):
    packed = _k1_call(R)
    out2 = _k2_call(dist_mat, packed)
    return out2[0, :_B], out2[1, :_B]


# speedup vs baseline: 3.7705x; 2.9457x over previous
import jax
import jax.numpy as jnp
from jax.experimental import pallas as pl
from jax.experimental.pallas import tpu as pltpu

_TOP_K = 10
_TAU = 1e-4
_EPS = 1e-8
_N = 2048
_B = 8
_CAP = 64
_TGT = 48
_BS_ITERS = 16
_NCHUNK = 4
_CHROWS = _N // _NCHUNK


def _shift_right(p, sh):
    z = jnp.zeros((p.shape[0], sh), p.dtype)
    return jnp.concatenate([z, p[:, : p.shape[1] - sh]], axis=1)


def _body(r_ref, m_hbm, out_ref, m_vmem, sems):
    cps = [
        pltpu.make_async_copy(
            m_hbm.at[pl.ds(k * _CHROWS, _CHROWS), :],
            m_vmem.at[pl.ds(k * _CHROWS, _CHROWS), :],
            sems.at[k],
        )
        for k in range(_NCHUNK)
    ]
    for cp in cps:
        cp.start()

    inv_tau = jnp.float32(1.0 / _TAU)
    x = r_ref[...]

    lo = jnp.min(x, axis=1, keepdims=True) - 1.0
    hi = jnp.max(x, axis=1, keepdims=True)

    def bs(_, carry):
        lo, hi = carry
        mid = 0.5 * (lo + hi)
        cnt = jnp.sum((x > mid).astype(jnp.float32), axis=1, keepdims=True)
        pred = cnt >= _TGT
        return jnp.where(pred, mid, lo), jnp.where(pred, hi, mid)

    lo, hi = jax.lax.fori_loop(0, _BS_ITERS, bs, (lo, hi))
    mask = (x > lo).astype(jnp.float32)

    p = mask
    sh = 1
    while sh < _N:
        p = p + _shift_right(p, sh)
        sh *= 2
    cnt_all = p[:, _N - 1 : _N]

    x_rep = jnp.concatenate(
        [jnp.broadcast_to(x[b : b + 1, :], (_CAP, _N)) for b in range(_B)], 0)
    p_rep = jnp.concatenate(
        [jnp.broadcast_to(p[b : b + 1, :], (_CAP, _N)) for b in range(_B)], 0)
    m_rep = jnp.concatenate(
        [jnp.broadcast_to(mask[b : b + 1, :], (_CAP, _N)) for b in range(_B)],
        0)
    cnt_rep = jnp.concatenate(
        [jnp.broadcast_to(cnt_all[b : b + 1, :], (_CAP, 1)) for b in range(_B)],
        0)
    nrows = _B * _CAP
    iota_r = jax.lax.broadcasted_iota(jnp.int32, (nrows, 1), 0)
    slot = (iota_r - (iota_r >> 6 << 6)).astype(jnp.float32)
    bt = jnp.where((p_rep - 1.0 == slot) & (m_rep > 0), 1.0, 0.0)
    cand = jnp.sum(bt * x_rep, axis=1, keepdims=True)
    d = (cand - x_rep) * inv_tau
    rank = jnp.sum(jax.nn.sigmoid(d), axis=1, keepdims=True) + 0.5
    valid = slot < cnt_rep
    wcol = jnp.where(valid, jax.nn.sigmoid(rank - (_N - _TOP_K)), 0.0)

    r_iota = jax.lax.broadcasted_iota(jnp.int32, (_B, nrows), 0)
    c_iota = jax.lax.broadcasted_iota(jnp.int32, (_B, nrows), 1)
    s_mat = jnp.where(c_iota >> 6 == r_iota, 1.0, 0.0)
    w_apx = jnp.dot(s_mat, wcol * bt, preferred_element_type=jnp.float32)

    iota_n = jax.lax.broadcasted_iota(jnp.int32, (_B, _N), 1)
    neg = jnp.float32(-jnp.inf)

    def kstep(_, carry):
        vals, ind = carry
        mx = jnp.max(vals, axis=1, keepdims=True)
        hit = vals == mx
        first = jnp.min(jnp.where(hit, iota_n, _N), axis=1, keepdims=True)
        oh = jnp.where(iota_n == first, 1.0, 0.0)
        return jnp.where(oh > 0, neg, vals), ind + oh

    _, ind = jax.lax.fori_loop(
        0, _TOP_K, kstep, (x, jnp.zeros((_B, _N), jnp.float32)))

    w = jnp.concatenate([w_apx, ind], axis=0)

    acc = jnp.zeros((2 * _B, _N), jnp.float32)
    for k in range(_NCHUNK):
        cps[k].wait()
        acc = acc + jnp.dot(
            w[:, k * _CHROWS : (k + 1) * _CHROWS],
            m_vmem[pl.ds(k * _CHROWS, _CHROWS), :],
            preferred_element_type=jnp.float32,
        )
    num = jnp.sum(acc * w, axis=1, keepdims=True)
    ws = jnp.sum(w, axis=1, keepdims=True)
    wss = jnp.sum(w * w, axis=1, keepdims=True)
    den = ws * ws - wss
    avg = num / (den + _EPS)
    avg = jnp.where(den == 0, 0.0, avg)
    out_ref[...] = jnp.broadcast_to(avg, (2 * _B, 128))


def kernel(R, dist_mat):
    out = pl.pallas_call(
        _body,
        in_specs=[
            pl.BlockSpec((_B, _N), lambda: (0, 0)),
            pl.BlockSpec(memory_space=pl.ANY),
        ],
        out_specs=pl.BlockSpec((2 * _B, 128), lambda: (0, 0)),
        out_shape=jax.ShapeDtypeStruct((2 * _B, 128), jnp.float32),
        scratch_shapes=[
            pltpu.VMEM((_N, _N), jnp.float32),
            pltpu.SemaphoreType.DMA((_NCHUNK,)),
        ],
    )(R, dist_mat)
    return out[:_B, 0], out[_B:, 0]


# speedup vs baseline: 3.8731x; 1.0272x over previous
import jax
import jax.numpy as jnp
from jax.experimental import pallas as pl
from jax.experimental.pallas import tpu as pltpu

_TOP_K = 10
_TAU = 1e-4
_EPS = 1e-8
_N = 2048
_B = 8
_CAP = 64
_TGT = 48
_BS_ITERS = 16
_NCHUNK = 1
_CHROWS = _N // _NCHUNK


def _shift_right(p, sh):
    z = jnp.zeros((p.shape[0], sh), p.dtype)
    return jnp.concatenate([z, p[:, : p.shape[1] - sh]], axis=1)


def _body(r_ref, m_hbm, out_ref, m_vmem, sems):
    cps = [
        pltpu.make_async_copy(
            m_hbm.at[pl.ds(k * _CHROWS, _CHROWS), :],
            m_vmem.at[pl.ds(k * _CHROWS, _CHROWS), :],
            sems.at[k],
        )
        for k in range(_NCHUNK)
    ]
    for cp in cps:
        cp.start()

    inv_tau = jnp.float32(1.0 / _TAU)
    x = r_ref[...]

    lo = jnp.min(x, axis=1, keepdims=True) - 1.0
    hi = jnp.max(x, axis=1, keepdims=True)

    def bs(_, carry):
        lo, hi = carry
        mid = 0.5 * (lo + hi)
        cnt = jnp.sum((x > mid).astype(jnp.float32), axis=1, keepdims=True)
        pred = cnt >= _TGT
        return jnp.where(pred, mid, lo), jnp.where(pred, hi, mid)

    lo, hi = jax.lax.fori_loop(0, _BS_ITERS, bs, (lo, hi))
    mask = (x > lo).astype(jnp.float32)

    p = mask
    sh = 1
    while sh < _N:
        p = p + _shift_right(p, sh)
        sh *= 2
    cnt_all = p[:, _N - 1 : _N]

    x_rep = jnp.concatenate(
        [jnp.broadcast_to(x[b : b + 1, :], (_CAP, _N)) for b in range(_B)], 0)
    p_rep = jnp.concatenate(
        [jnp.broadcast_to(p[b : b + 1, :], (_CAP, _N)) for b in range(_B)], 0)
    m_rep = jnp.concatenate(
        [jnp.broadcast_to(mask[b : b + 1, :], (_CAP, _N)) for b in range(_B)],
        0)
    cnt_rep = jnp.concatenate(
        [jnp.broadcast_to(cnt_all[b : b + 1, :], (_CAP, 1)) for b in range(_B)],
        0)
    nrows = _B * _CAP
    iota_r = jax.lax.broadcasted_iota(jnp.int32, (nrows, 1), 0)
    slot = (iota_r - (iota_r >> 6 << 6)).astype(jnp.float32)
    bt = jnp.where((p_rep - 1.0 == slot) & (m_rep > 0), 1.0, 0.0)
    cand = jnp.sum(bt * x_rep, axis=1, keepdims=True)
    d = (cand - x_rep) * inv_tau
    rank = jnp.sum(jax.nn.sigmoid(d), axis=1, keepdims=True) + 0.5
    valid = slot < cnt_rep
    wcol = jnp.where(valid, jax.nn.sigmoid(rank - (_N - _TOP_K)), 0.0)

    r_iota = jax.lax.broadcasted_iota(jnp.int32, (_B, nrows), 0)
    c_iota = jax.lax.broadcasted_iota(jnp.int32, (_B, nrows), 1)
    s_mat = jnp.where(c_iota >> 6 == r_iota, 1.0, 0.0)
    w_apx = jnp.dot(s_mat, wcol * bt, preferred_element_type=jnp.float32)

    iota_n = jax.lax.broadcasted_iota(jnp.int32, (_B, _N), 1)
    neg = jnp.float32(-jnp.inf)

    def kstep(_, carry):
        vals, ind = carry
        mx = jnp.max(vals, axis=1, keepdims=True)
        hit = vals == mx
        first = jnp.min(jnp.where(hit, iota_n, _N), axis=1, keepdims=True)
        oh = jnp.where(iota_n == first, 1.0, 0.0)
        return jnp.where(oh > 0, neg, vals), ind + oh

    _, ind = jax.lax.fori_loop(
        0, _TOP_K, kstep, (x, jnp.zeros((_B, _N), jnp.float32)))

    w = jnp.concatenate([w_apx, ind], axis=0)

    acc = jnp.zeros((2 * _B, _N), jnp.float32)
    for k in range(_NCHUNK):
        cps[k].wait()
        acc = acc + jnp.dot(
            w[:, k * _CHROWS : (k + 1) * _CHROWS],
            m_vmem[pl.ds(k * _CHROWS, _CHROWS), :],
            preferred_element_type=jnp.float32,
        )
    num = jnp.sum(acc * w, axis=1, keepdims=True)
    ws = jnp.sum(w, axis=1, keepdims=True)
    wss = jnp.sum(w * w, axis=1, keepdims=True)
    den = ws * ws - wss
    avg = num / (den + _EPS)
    avg = jnp.where(den == 0, 0.0, avg)
    out_ref[...] = jnp.broadcast_to(avg, (2 * _B, 128))


def kernel(R, dist_mat):
    out = pl.pallas_call(
        _body,
        in_specs=[
            pl.BlockSpec((_B, _N), lambda: (0, 0)),
            pl.BlockSpec(memory_space=pl.ANY),
        ],
        out_specs=pl.BlockSpec((2 * _B, 128), lambda: (0, 0)),
        out_shape=jax.ShapeDtypeStruct((2 * _B, 128), jnp.float32),
        scratch_shapes=[
            pltpu.VMEM((_N, _N), jnp.float32),
            pltpu.SemaphoreType.DMA((_NCHUNK,)),
        ],
    )(R, dist_mat)
    return out[:_B, 0], out[_B:, 0]
